# baseline (speedup 1.0000x reference)
import jax
import jax.numpy as jnp
from jax.experimental import pallas as pl
from jax.experimental.pallas import tpu as pltpu

D_MODEL = 4096
NUM_E = 64
N_TOK = 4 * 4096
TOK_BLK = 1024
GRID = N_TOK // TOK_BLK


def _router_body(x_ref, w_ref, b_ref, top1_ref, prob_ref, aux_ref,
                 lg_ref, acc_ref):
    i = pl.program_id(0)
    cur = jax.lax.rem(i, 2)
    prv = 1 - cur

    logits = lg_ref[prv]
    m = jnp.max(logits, axis=0, keepdims=True)
    e = jnp.exp(logits - m)
    s = jnp.sum(e, axis=0, keepdims=True)
    rs = 1.0 / s
    top1 = jnp.argmax(logits, axis=0).astype(jnp.int32)
    top1_ref[0, 0, :] = top1
    prob_ref[0, 0, :] = rs[0, :]

    probs = e * rs
    imp_part = jnp.sum(probs, axis=1)
    iota = jax.lax.broadcasted_iota(jnp.int32, (NUM_E, TOK_BLK), 0)
    cnt_part = jnp.sum((iota == top1[None, :]).astype(jnp.float32), axis=1)
    part = jnp.concatenate([imp_part[None, :], cnt_part[None, :]], axis=0)

    acc = jnp.where(i > 1, acc_ref[...], 0.0) + jnp.where(i > 0, part, 0.0)
    acc_ref[...] = acc
    aux_ref[...] = (NUM_E / (N_TOK * N_TOK)) * jnp.sum(
        acc[0:1, :] * acc[1:2, :], axis=1, keepdims=True)

    lg_ref[cur] = jax.lax.dot_general(
        w_ref[...], x_ref[...], (((1,), (1,)), ((), ())),
        preferred_element_type=jnp.float32) + b_ref[...]


def kernel(x, W, b):
    xf = x.reshape(N_TOK, D_MODEL)
    b2 = b.reshape(NUM_E, 1)
    top1, prob, aux = pl.pallas_call(
        _router_body,
        grid=(GRID + 1,),
        in_specs=[
            pl.BlockSpec((TOK_BLK, D_MODEL),
                         lambda i: (jnp.minimum(i, GRID - 1), 0)),
            pl.BlockSpec((NUM_E, D_MODEL), lambda i: (0, 0)),
            pl.BlockSpec((NUM_E, 1), lambda i: (0, 0)),
        ],
        out_specs=[
            pl.BlockSpec((1, 1, TOK_BLK),
                         lambda i: (jnp.maximum(i - 1, 0), 0, 0)),
            pl.BlockSpec((1, 1, TOK_BLK),
                         lambda i: (jnp.maximum(i - 1, 0), 0, 0)),
            pl.BlockSpec((1, 1), lambda i: (0, 0)),
        ],
        out_shape=[
            jax.ShapeDtypeStruct((GRID, 1, TOK_BLK), jnp.int32),
            jax.ShapeDtypeStruct((GRID, 1, TOK_BLK), jnp.float32),
            jax.ShapeDtypeStruct((1, 1), jnp.float32),
        ],
        scratch_shapes=[
            pltpu.VMEM((2, NUM_E, TOK_BLK), jnp.float32),
            pltpu.VMEM((2, NUM_E), jnp.float32),
        ],
        compiler_params=pltpu.CompilerParams(
            dimension_semantics=("arbitrary",),
        ),
    )(xf, W, b2)
    return (top1.reshape(x.shape[0], x.shape[1]),
            prob.reshape(x.shape[0], x.shape[1]),
            aux.reshape(()))


# speedup vs baseline: 1.0355x; 1.0355x over previous
import jax
import jax.numpy as jnp
from jax.experimental import pallas as pl
from jax.experimental.pallas import tpu as pltpu

D_MODEL = 4096
NUM_E = 64
N_TOK = 4 * 4096
TOK_BLK = 1024
GRID = N_TOK // TOK_BLK


def _router_body(x_ref, w_ref, b_ref, top1_ref, prob_ref, aux_ref, acc_ref):
    i = pl.program_id(0)
    logits = jax.lax.dot_general(
        w_ref[...], x_ref[...], (((1,), (1,)), ((), ())),
        preferred_element_type=jnp.float32) + b_ref[...]
    m = jnp.max(logits, axis=0, keepdims=True)
    e = jnp.exp(logits - m)
    s = jnp.sum(e, axis=0, keepdims=True)
    rs = 1.0 / s
    top1 = jnp.argmax(logits, axis=0).astype(jnp.int32)
    top1_ref[0, 0, :] = top1
    prob_ref[0, 0, :] = rs[0, :]

    probs = e * rs
    imp_part = jnp.sum(probs, axis=1)
    iota = jax.lax.broadcasted_iota(jnp.int32, (NUM_E, TOK_BLK), 0)
    cnt_part = jnp.sum((iota == top1[None, :]).astype(jnp.float32), axis=1)
    part = jnp.concatenate([imp_part[None, :], cnt_part[None, :]], axis=0)

    acc = jnp.where(i > 0, acc_ref[...], 0.0) + part
    acc_ref[...] = acc
    aux_ref[...] = (NUM_E / (N_TOK * N_TOK)) * jnp.sum(
        acc[0:1, :] * acc[1:2, :], axis=1, keepdims=True)


def kernel(x, W, b):
    xf = x.reshape(N_TOK, D_MODEL)
    b2 = b.reshape(NUM_E, 1)
    top1, prob, aux = pl.pallas_call(
        _router_body,
        grid=(GRID,),
        in_specs=[
            pl.BlockSpec((TOK_BLK, D_MODEL), lambda i: (i, 0)),
            pl.BlockSpec((NUM_E, D_MODEL), lambda i: (0, 0)),
            pl.BlockSpec((NUM_E, 1), lambda i: (0, 0)),
        ],
        out_specs=[
            pl.BlockSpec((1, 1, TOK_BLK), lambda i: (i, 0, 0)),
            pl.BlockSpec((1, 1, TOK_BLK), lambda i: (i, 0, 0)),
            pl.BlockSpec((1, 1), lambda i: (0, 0)),
        ],
        out_shape=[
            jax.ShapeDtypeStruct((GRID, 1, TOK_BLK), jnp.int32),
            jax.ShapeDtypeStruct((GRID, 1, TOK_BLK), jnp.float32),
            jax.ShapeDtypeStruct((1, 1), jnp.float32),
        ],
        scratch_shapes=[pltpu.VMEM((2, NUM_E), jnp.float32)],
        compiler_params=pltpu.CompilerParams(
            dimension_semantics=("arbitrary",),
        ),
    )(xf, W, b2)
    return (top1.reshape(x.shape[0], x.shape[1]),
            prob.reshape(x.shape[0], x.shape[1]),
            aux.reshape(()))
